# Initial kernel scaffold; baseline (speedup 1.0000x reference)
import jax
import jax.numpy as jnp
from jax.experimental import pallas as pl


def kernel(x, y, x_pos, y_pos):
    raise NotImplementedError("write your pallas kernel here")


# speedup vs baseline: 4556.4782x; 4556.4782x over previous
import jax
import jax.numpy as jnp
from jax import lax
from jax.experimental import pallas as pl
from jax.experimental.pallas import tpu as pltpu
from jax.experimental.pallas import tpu_sc as plsc

L = 16
NC = 2
NS = 16
NW = NC * NS

QBITS = 18
IDXB = 12
DIGB = 6
NDIG = 1 << DIGB


def _w1_body(x_hbm, y_hbm, xp_hbm, yp_hbm, out_hbm,
             wbufs, posbufs, pka, pkb, h0, h1, resbuf, sems):
  B, N = x_hbm.shape
  M = y_hbm.shape[1]
  E = N + M
  VREGS = E // L
  BLK = E // (2 * L)
  rows_per_w = B // NW

  cid = lax.axis_index("c")
  sid = lax.axis_index("s")
  wid = sid * NC + cid
  row0 = wid * rows_per_w

  iota = lax.iota(jnp.int32, L)
  ones = jnp.ones((L,), jnp.int32)
  gbase0 = iota * BLK
  gbase1 = (iota + L) * BLK
  idx_mask = (1 << IDXB) - 1

  def start_load(row, b):
    pltpu.async_copy(x_hbm.at[row], wbufs[b].at[pl.ds(0, N)], sems[b])
    pltpu.async_copy(y_hbm.at[row], wbufs[b].at[pl.ds(N, M)], sems[b])
    pltpu.async_copy(xp_hbm.at[row], posbufs[b].at[pl.ds(0, N)], sems[b])
    pltpu.async_copy(yp_hbm.at[row], posbufs[b].at[pl.ds(N, M)], sems[b])

  def wait_load(row, b):
    pltpu.make_async_copy(x_hbm.at[row], wbufs[b].at[pl.ds(0, N)], sems[b]).wait()
    pltpu.make_async_copy(y_hbm.at[row], wbufs[b].at[pl.ds(N, M)], sems[b]).wait()
    pltpu.make_async_copy(xp_hbm.at[row], posbufs[b].at[pl.ds(0, N)], sems[b]).wait()
    pltpu.make_async_copy(yp_hbm.at[row], posbufs[b].at[pl.ds(N, M)], sems[b]).wait()

  def radix_pass(src, dst, shift):
    def zero_body(d, _):
      h0[pl.ds(d * L, L)] = jnp.zeros((L,), jnp.int32)
      h1[pl.ds(d * L, L)] = jnp.zeros((L,), jnp.int32)
      return 0
    lax.fori_loop(0, NDIG, zero_body, 0, unroll=8)

    def hist_body(v, _):
      p0 = plsc.load_gather(src, [gbase0 + v])
      d0 = lax.shift_right_logical(p0, shift) & (NDIG - 1)
      plsc.addupdate_scatter(h0, [d0 * L + iota], ones)
      p1 = plsc.load_gather(src, [gbase1 + v])
      d1 = lax.shift_right_logical(p1, shift) & (NDIG - 1)
      plsc.addupdate_scatter(h1, [d1 * L + iota], ones)
      return 0
    lax.fori_loop(0, BLK, hist_body, 0, unroll=4)

    def scan_body(d, carry):
      hv0 = h0[pl.ds(d * L, L)]
      hv1 = h1[pl.ds(d * L, L)]
      cs0 = plsc.cumsum(hv0)
      cs1 = plsc.cumsum(hv1)
      t0 = jnp.sum(hv0)
      h0[pl.ds(d * L, L)] = carry + (cs0 - hv0)
      h1[pl.ds(d * L, L)] = (carry + t0) + (cs1 - hv1)
      return carry + t0 + jnp.sum(hv1)
    lax.fori_loop(0, NDIG, scan_body, jnp.int32(0), unroll=8)

    def perm_body(v, _):
      p0 = plsc.load_gather(src, [gbase0 + v])
      d0 = lax.shift_right_logical(p0, shift) & (NDIG - 1)
      b0 = d0 * L + iota
      o0 = plsc.load_gather(h0, [b0])
      plsc.store_scatter(dst, [o0], p0)
      plsc.store_scatter(h0, [b0], o0 + 1)
      p1 = plsc.load_gather(src, [gbase1 + v])
      d1 = lax.shift_right_logical(p1, shift) & (NDIG - 1)
      b1 = d1 * L + iota
      o1 = plsc.load_gather(h1, [b1])
      plsc.store_scatter(dst, [o1], p1)
      plsc.store_scatter(h1, [b1], o1 + 1)
      return 0
    lax.fori_loop(0, BLK, perm_body, 0, unroll=2)

  qmax = jnp.int32((1 << QBITS) - 1)
  qscale = jnp.float32(1 << QBITS)

  def compute_row(r, b):
    wbuf = wbufs[b]
    posbuf = posbufs[b]

    def sum_body(i, accs):
      ax, ay = accs
      return (ax + wbuf[pl.ds(i * L, L)], ay + wbuf[pl.ds(N + i * L, L)])
    ax, ay = lax.fori_loop(0, N // L, sum_body,
                           (jnp.zeros((L,), jnp.float32),
                            jnp.zeros((L,), jnp.float32)), unroll=4)
    gx = 1.0 / jnp.full((L,), jnp.sum(ax), jnp.float32)
    gy = -1.0 / jnp.full((L,), jnp.sum(ay), jnp.float32)

    def build_body(i, _):
      pos = posbuf[pl.ds(i * L, L)]
      q = jnp.minimum((pos * qscale).astype(jnp.int32), qmax)
      pka[pl.ds(i * L, L)] = q * (1 << IDXB) + (i * L + iota)
      return 0
    lax.fori_loop(0, VREGS, build_body, 0, unroll=4)

    radix_pass(pka, pkb, IDXB)
    radix_pass(pkb, pka, IDXB + DIGB)
    radix_pass(pka, pkb, IDXB + 2 * DIGB)

    plast = pkb[pl.ds(E - L, L)]
    plsc.store_scatter(pkb, [jnp.full((L,), E, jnp.int32)], plast,
                       mask=iota == L - 1)

    def fin_body(v, carry):
      csum, acc = carry
      p = pkb[pl.ds(v * L, L)]
      pn = pkb[pl.ds(v * L + 1, L)]
      i0 = p & idx_mask
      i1 = pn & idx_mask
      pos = plsc.load_gather(posbuf, [i0])
      posn = plsc.load_gather(posbuf, [i1])
      wraw = plsc.load_gather(wbuf, [i0])
      w = wraw * jnp.where(i0 < N, gx, gy)
      cum = csum + plsc.cumsum(w)
      acc = acc + (posn - pos) * jnp.abs(cum)
      return (csum + jnp.sum(w), acc)
    _, acc = lax.fori_loop(0, VREGS, fin_body,
                           (jnp.float32(0.0), jnp.zeros((L,), jnp.float32)),
                           unroll=2)
    res = jnp.sum(acc)
    plsc.store_scatter(resbuf, [jnp.full((L,), r, jnp.int32)],
                       jnp.full((L,), res, jnp.float32), mask=iota == 0)

  start_load(row0, 0)

  def do_pair(pr, _):
    ra = 2 * pr
    rb = 2 * pr + 1
    start_load(row0 + rb, 1)
    wait_load(row0 + ra, 0)
    compute_row(ra, 0)
    rnext = (ra + 2) & (rows_per_w - 1)
    start_load(row0 + rnext, 0)
    wait_load(row0 + rb, 1)
    compute_row(rb, 1)
    return 0
  lax.fori_loop(0, rows_per_w // 2, do_pair, 0)
  wait_load(row0, 0)

  pltpu.sync_copy(resbuf, out_hbm.at[pl.ds(row0, rows_per_w)])


def kernel(x, y, x_pos, y_pos):
  B, N = x.shape
  M = y.shape[1]
  E = N + M
  fn = pl.kernel(
      _w1_body,
      out_type=jax.ShapeDtypeStruct((B,), jnp.float32),
      mesh=plsc.VectorSubcoreMesh(core_axis_name="c", subcore_axis_name="s"),
      compiler_params=pltpu.CompilerParams(needs_layout_passes=False),
      scratch_types=[
          [pltpu.VMEM((E,), jnp.float32)] * 2,
          [pltpu.VMEM((E,), jnp.float32)] * 2,
          pltpu.VMEM((E + L,), jnp.int32),
          pltpu.VMEM((E + L,), jnp.int32),
          pltpu.VMEM((NDIG * L,), jnp.int32),
          pltpu.VMEM((NDIG * L,), jnp.int32),
          pltpu.VMEM((B // NW,), jnp.float32),
          [pltpu.SemaphoreType.DMA] * 2,
      ],
  )
  return fn(x, y, x_pos, y_pos)
